# Initial kernel scaffold; baseline (speedup 1.0000x reference)
import jax
import jax.numpy as jnp
from jax.experimental import pallas as pl


def kernel(x, edge_index, W1, b1, Wh, bh, W2, b2):
    raise NotImplementedError("write your pallas kernel here")


# speedup vs baseline: 10.4046x; 10.4046x over previous
import functools

import jax
import jax.numpy as jnp
from jax import lax
from jax.experimental import pallas as pl
from jax.experimental.pallas import tpu as pltpu
from jax.experimental.pallas import tpu_sc as plsc

N = 10000
E = 320000
D = 128
DD = 16
NC = 2
NS = 16
NW = NC * NS
K = 80
EPT = E // NW
NCH = EPT // K
NP = 12800
RPT = NP // NS
RB = 400
NH = 10240

def _zero_rows(buf, acc, base):
    for k in range(RPT // K):
        pltpu.sync_copy(buf, acc.at[pl.ds(base + k * K, K)])


@functools.cache
def _agg_call():
    mesh = plsc.VectorSubcoreMesh(core_axis_name="c", subcore_axis_name="s")

    @functools.partial(
        pl.kernel,
        mesh=mesh,
        out_type=jax.ShapeDtypeStruct((2 * NP, D), jnp.float32),
        scratch_types=[
            pltpu.VMEM((K,), jnp.int32),
            pltpu.VMEM((K,), jnp.int32),
            pltpu.VMEM((K, D), jnp.float32),
            pltpu.VMEM_SHARED((NP, D), jnp.float32),
            pltpu.SemaphoreType.DMA,
        ],
    )
    def agg(y_hbm, src_hbm, dst_hbm, z_hbm, out_hbm, src_v, dst_v, rows_v, acc, sem):
        c = lax.axis_index("c")
        s = lax.axis_index("s")
        wid = s * NC + c
        base = s * RPT
        pltpu.sync_copy(z_hbm, rows_v)
        _zero_rows(rows_v, acc, base)
        plsc.subcore_barrier()

        def body(j, carry):
            off = wid * EPT + j * K
            pltpu.sync_copy(src_hbm.at[pl.ds(off, K)], src_v)
            pltpu.sync_copy(dst_hbm.at[pl.ds(off, K)], dst_v)
            pltpu.async_copy(y_hbm.at[src_v], rows_v, sem).wait()
            pltpu.sync_copy(rows_v, acc.at[dst_v], add=True)
            return carry

        lax.fori_loop(0, NCH, body, 0, unroll=False)
        plsc.subcore_barrier()
        pltpu.sync_copy(acc.at[pl.ds(base, RPT)], out_hbm.at[pl.ds(c * NP + base, RPT)])

    return agg


@functools.cache
def _deg_call():
    mesh = plsc.VectorSubcoreMesh(core_axis_name="c", subcore_axis_name="s")

    @functools.partial(
        pl.kernel,
        mesh=mesh,
        out_type=jax.ShapeDtypeStruct((2 * NP, D), jnp.float32),
        scratch_types=[
            pltpu.VMEM((K,), jnp.int32),
            pltpu.VMEM((K, D), jnp.float32),
            pltpu.VMEM_SHARED((NP, D), jnp.float32),
        ],
    )
    def deg(dst_hbm, ones_hbm, z_hbm, out_hbm, dst_v, rows_v, acc):
        c = lax.axis_index("c")
        s = lax.axis_index("s")
        wid = s * NC + c
        base = s * RPT
        pltpu.sync_copy(z_hbm, rows_v)
        _zero_rows(rows_v, acc, base)
        pltpu.sync_copy(ones_hbm, rows_v)
        plsc.subcore_barrier()

        def body(j, carry):
            off = wid * EPT + j * K
            pltpu.sync_copy(dst_hbm.at[pl.ds(off, K)], dst_v)
            pltpu.sync_copy(rows_v, acc.at[dst_v], add=True)
            return carry

        lax.fori_loop(0, NCH, body, 0, unroll=False)
        plsc.subcore_barrier()
        pltpu.sync_copy(acc.at[pl.ds(base, RPT)], out_hbm.at[pl.ds(c * NP + base, RPT)])

    return deg


def _k1_body(x_ref, w_ref, dp0_ref, dp1_ref, y_ref, dinv_ref):
    deg = dp0_ref[...][:, 0:1] + dp1_ref[...][:, 0:1] + 1.0
    dinv = lax.rsqrt(jnp.maximum(deg, 1.0))
    xw = jnp.dot(x_ref[...], w_ref[...], preferred_element_type=jnp.float32)
    dinv_ref[...] = jnp.broadcast_to(dinv, (RB, DD))
    y_ref[...] = xw * dinv


def _k1(x, W1, dp):
    grid = (N // RB,)
    return pl.pallas_call(
        _k1_body,
        grid=grid,
        in_specs=[
            pl.BlockSpec((RB, D), lambda i: (i, 0)),
            pl.BlockSpec((D, D), lambda i: (0, 0)),
            pl.BlockSpec((RB, D), lambda i: (i, 0)),
            pl.BlockSpec((RB, D), lambda i: (i + NP // RB, 0)),
        ],
        out_specs=[
            pl.BlockSpec((RB, D), lambda i: (i, 0)),
            pl.BlockSpec((RB, DD), lambda i: (i, 0)),
        ],
        out_shape=[
            jax.ShapeDtypeStruct((N, D), jnp.float32),
            jax.ShapeDtypeStruct((N, DD), jnp.float32),
        ],
    )(x, W1, dp, dp)


def _k2_body(p0_ref, p1_ref, yp_ref, dinv_ref, b_ref, w_ref, yn_ref):
    di = dinv_ref[...][:, 0:1]
    z = (p0_ref[...] + p1_ref[...] + yp_ref[...]) * di + b_ref[...]
    h = jnp.maximum(z, 0.0)
    yn_ref[...] = jnp.dot(h, w_ref[...], preferred_element_type=jnp.float32) * di


def _k2(p, yp, dinv8, b, W):
    grid = (N // RB,)
    return pl.pallas_call(
        _k2_body,
        grid=grid,
        in_specs=[
            pl.BlockSpec((RB, D), lambda i: (i, 0)),
            pl.BlockSpec((RB, D), lambda i: (i + NP // RB, 0)),
            pl.BlockSpec((RB, D), lambda i: (i, 0)),
            pl.BlockSpec((RB, DD), lambda i: (i, 0)),
            pl.BlockSpec((1, D), lambda i: (0, 0)),
            pl.BlockSpec((D, D), lambda i: (0, 0)),
        ],
        out_specs=pl.BlockSpec((RB, D), lambda i: (i, 0)),
        out_shape=jax.ShapeDtypeStruct((N, D), jnp.float32),
    )(p, p, yp, dinv8, b.reshape(1, D), W)


def _k3_body(p0_ref, p1_ref, yp_ref, dinv_ref, b_ref, o_ref):
    di = dinv_ref[...][:, 0:1]
    z = (p0_ref[...] + p1_ref[...] + yp_ref[...]) * di + b_ref[...]
    m = jnp.max(z, axis=1, keepdims=True)
    e = jnp.exp(z - m)
    o_ref[...] = z - m - jnp.log(jnp.sum(e, axis=1, keepdims=True))


def _k3(p, yp, dinv8, b):
    grid = (N // RB,)
    return pl.pallas_call(
        _k3_body,
        grid=grid,
        in_specs=[
            pl.BlockSpec((RB, D), lambda i: (i, 0)),
            pl.BlockSpec((RB, D), lambda i: (i + NP // RB, 0)),
            pl.BlockSpec((RB, D), lambda i: (i, 0)),
            pl.BlockSpec((RB, DD), lambda i: (i, 0)),
            pl.BlockSpec((1, D), lambda i: (0, 0)),
        ],
        out_specs=pl.BlockSpec((RB, D), lambda i: (i, 0)),
        out_shape=jax.ShapeDtypeStruct((N, D), jnp.float32),
    )(p, p, yp, dinv8, b.reshape(1, D))


def kernel(x, edge_index, W1, b1, Wh, bh, W2, b2):
    src = edge_index[0].astype(jnp.int32)
    dst = edge_index[1].astype(jnp.int32)
    zeros_d = jnp.zeros((K, D), jnp.float32)
    ones_d = jnp.ones((K, D), jnp.float32)

    dp = _deg_call()(dst, ones_d, zeros_d)
    y1, dinv8 = _k1(x, W1, dp)
    p = _agg_call()(y1, src, dst, zeros_d)
    y2 = _k2(p, y1, dinv8, b1, Wh)
    p = _agg_call()(y2, src, dst, zeros_d)
    y3 = _k2(p, y2, dinv8, bh, W2)
    p = _agg_call()(y3, src, dst, zeros_d)
    return _k3(p, y3, dinv8, b2)


# speedup vs baseline: 14.7345x; 1.4162x over previous
import functools

import jax
import jax.numpy as jnp
from jax import lax
from jax.experimental import pallas as pl
from jax.experimental.pallas import tpu as pltpu
from jax.experimental.pallas import tpu_sc as plsc

N = 10000
E = 320000
D = 128
DD = 16
NC = 2
NS = 16
NW = NC * NS
K = 40
EPT = E // NW
NCH = EPT // K
G = 2
NG = NCH // G
NP = 12800
RPT = NP // NS
RB = 400


def _zero_rows(buf, acc, base):
    for k in range(RPT // K):
        pltpu.sync_copy(buf, acc.at[pl.ds(base + k * K, K)])


@functools.cache
def _agg_call():
    mesh = plsc.VectorSubcoreMesh(core_axis_name="c", subcore_axis_name="s")
    scratch = (
        [pltpu.VMEM((K,), jnp.int32) for _ in range(2 * G)]
        + [pltpu.VMEM((K,), jnp.int32) for _ in range(2 * G)]
        + [pltpu.VMEM((K, D), jnp.float32) for _ in range(2 * G)]
        + [pltpu.VMEM_SHARED((NP, D), jnp.float32)]
        + [pltpu.SemaphoreType.DMA] * 6
    )

    @functools.partial(
        pl.kernel,
        mesh=mesh,
        out_type=jax.ShapeDtypeStruct((2 * NP, D), jnp.float32),
        scratch_types=scratch,
    )
    def agg(y_hbm, src_hbm, dst_hbm, z_hbm, out_hbm, *scr):
        srcs = scr[0:2 * G]
        dsts = scr[2 * G:4 * G]
        rows = scr[4 * G:6 * G]
        acc = scr[6 * G]
        isems = scr[6 * G + 1:6 * G + 3]
        gsems = scr[6 * G + 3:6 * G + 5]
        ssems = scr[6 * G + 5:6 * G + 7]
        c = lax.axis_index("c")
        s = lax.axis_index("s")
        wid = s * NC + c
        base = s * RPT
        ebase = wid * EPT

        pltpu.sync_copy(z_hbm, rows[0])
        _zero_rows(rows[0], acc, base)
        plsc.subcore_barrier()

        def fire_group(g, P):
            for i in range(G):
                off = ebase + (g * G + i) * K
                pltpu.async_copy(src_hbm.at[pl.ds(off, K)], srcs[P * G + i], isems[P])
                pltpu.async_copy(dst_hbm.at[pl.ds(off, K)], dsts[P * G + i], isems[P])
            for i in range(G):
                pltpu.make_async_copy(src_hbm.at[pl.ds(0, K)], srcs[P * G + i], isems[P]).wait()
                pltpu.make_async_copy(dst_hbm.at[pl.ds(0, K)], dsts[P * G + i], isems[P]).wait()
            for i in range(G):
                pltpu.async_copy(y_hbm.at[srcs[P * G + i]], rows[P * G + i], gsems[P])
            for i in range(G):
                pltpu.make_async_copy(y_hbm.at[pl.ds(0, K)], rows[P * G + i], gsems[P]).wait()
            for i in range(G):
                pltpu.async_copy(rows[P * G + i], acc.at[dsts[P * G + i]], ssems[P], add=True)

        def drain_scatters(P):
            for i in range(G):
                pltpu.make_async_copy(y_hbm.at[pl.ds(0, K)], rows[P * G + i], ssems[P]).wait()

        fire_group(0, 0)
        fire_group(1, 1)

        def body(gg, carry):
            drain_scatters(0)
            fire_group(2 * gg, 0)
            drain_scatters(1)
            fire_group(2 * gg + 1, 1)
            return carry

        lax.fori_loop(1, NG // 2, body, 0, unroll=False)
        drain_scatters(0)
        fire_group(NG - 1, 0)
        drain_scatters(1)
        drain_scatters(0)
        plsc.subcore_barrier()
        pltpu.sync_copy(acc.at[pl.ds(base, RPT)], out_hbm.at[pl.ds(c * NP + base, RPT)])

    return agg


@functools.cache
def _deg_call():
    mesh = plsc.VectorSubcoreMesh(core_axis_name="c", subcore_axis_name="s")
    scratch = (
        [pltpu.VMEM((K,), jnp.int32) for _ in range(2 * G)]
        + [pltpu.VMEM((K, D), jnp.float32)]
        + [pltpu.VMEM_SHARED((NP, D), jnp.float32)]
        + [pltpu.SemaphoreType.DMA] * 4
    )

    @functools.partial(
        pl.kernel,
        mesh=mesh,
        out_type=jax.ShapeDtypeStruct((2 * NP, D), jnp.float32),
        scratch_types=scratch,
    )
    def deg(dst_hbm, ones_hbm, z_hbm, out_hbm, *scr):
        dsts = scr[0:2 * G]
        ones_v = scr[2 * G]
        acc = scr[2 * G + 1]
        isems = scr[2 * G + 2:2 * G + 4]
        ssems = scr[2 * G + 4:2 * G + 6]
        c = lax.axis_index("c")
        s = lax.axis_index("s")
        wid = s * NC + c
        base = s * RPT
        ebase = wid * EPT

        pltpu.sync_copy(z_hbm, ones_v)
        _zero_rows(ones_v, acc, base)
        pltpu.sync_copy(ones_hbm, ones_v)
        plsc.subcore_barrier()

        def fire_group(g, P):
            for i in range(G):
                off = ebase + (g * G + i) * K
                pltpu.async_copy(dst_hbm.at[pl.ds(off, K)], dsts[P * G + i], isems[P])
            for i in range(G):
                pltpu.make_async_copy(dst_hbm.at[pl.ds(0, K)], dsts[P * G + i], isems[P]).wait()
            for i in range(G):
                pltpu.async_copy(ones_v, acc.at[dsts[P * G + i]], ssems[P], add=True)

        def drain_scatters(P):
            for _ in range(G):
                pltpu.make_async_copy(ones_hbm, ones_v, ssems[P]).wait()

        fire_group(0, 0)
        fire_group(1, 1)

        def body(gg, carry):
            drain_scatters(0)
            fire_group(2 * gg, 0)
            drain_scatters(1)
            fire_group(2 * gg + 1, 1)
            return carry

        lax.fori_loop(1, NG // 2, body, 0, unroll=False)
        drain_scatters(0)
        fire_group(NG - 1, 0)
        drain_scatters(1)
        drain_scatters(0)
        plsc.subcore_barrier()
        pltpu.sync_copy(acc.at[pl.ds(base, RPT)], out_hbm.at[pl.ds(c * NP + base, RPT)])

    return deg


def _k1_body(x_ref, w_ref, dp0_ref, dp1_ref, y_ref, dinv_ref):
    deg = dp0_ref[...][:, 0:1] + dp1_ref[...][:, 0:1] + 1.0
    dinv = lax.rsqrt(jnp.maximum(deg, 1.0))
    xw = jnp.dot(x_ref[...], w_ref[...], preferred_element_type=jnp.float32)
    dinv_ref[...] = jnp.broadcast_to(dinv, (RB, DD))
    y_ref[...] = xw * dinv


def _k1(x, W1, dp):
    grid = (N // RB,)
    return pl.pallas_call(
        _k1_body,
        grid=grid,
        in_specs=[
            pl.BlockSpec((RB, D), lambda i: (i, 0)),
            pl.BlockSpec((D, D), lambda i: (0, 0)),
            pl.BlockSpec((RB, D), lambda i: (i, 0)),
            pl.BlockSpec((RB, D), lambda i: (i + NP // RB, 0)),
        ],
        out_specs=[
            pl.BlockSpec((RB, D), lambda i: (i, 0)),
            pl.BlockSpec((RB, DD), lambda i: (i, 0)),
        ],
        out_shape=[
            jax.ShapeDtypeStruct((N, D), jnp.float32),
            jax.ShapeDtypeStruct((N, DD), jnp.float32),
        ],
    )(x, W1, dp, dp)


def _k2_body(p0_ref, p1_ref, yp_ref, dinv_ref, b_ref, w_ref, yn_ref):
    di = dinv_ref[...][:, 0:1]
    z = (p0_ref[...] + p1_ref[...] + yp_ref[...]) * di + b_ref[...]
    h = jnp.maximum(z, 0.0)
    yn_ref[...] = jnp.dot(h, w_ref[...], preferred_element_type=jnp.float32) * di


def _k2(p, yp, dinv8, b, W):
    grid = (N // RB,)
    return pl.pallas_call(
        _k2_body,
        grid=grid,
        in_specs=[
            pl.BlockSpec((RB, D), lambda i: (i, 0)),
            pl.BlockSpec((RB, D), lambda i: (i + NP // RB, 0)),
            pl.BlockSpec((RB, D), lambda i: (i, 0)),
            pl.BlockSpec((RB, DD), lambda i: (i, 0)),
            pl.BlockSpec((1, D), lambda i: (0, 0)),
            pl.BlockSpec((D, D), lambda i: (0, 0)),
        ],
        out_specs=pl.BlockSpec((RB, D), lambda i: (i, 0)),
        out_shape=jax.ShapeDtypeStruct((N, D), jnp.float32),
    )(p, p, yp, dinv8, b.reshape(1, D), W)


def _k3_body(p0_ref, p1_ref, yp_ref, dinv_ref, b_ref, o_ref):
    di = dinv_ref[...][:, 0:1]
    z = (p0_ref[...] + p1_ref[...] + yp_ref[...]) * di + b_ref[...]
    m = jnp.max(z, axis=1, keepdims=True)
    e = jnp.exp(z - m)
    o_ref[...] = z - m - jnp.log(jnp.sum(e, axis=1, keepdims=True))


def _k3(p, yp, dinv8, b):
    grid = (N // RB,)
    return pl.pallas_call(
        _k3_body,
        grid=grid,
        in_specs=[
            pl.BlockSpec((RB, D), lambda i: (i, 0)),
            pl.BlockSpec((RB, D), lambda i: (i + NP // RB, 0)),
            pl.BlockSpec((RB, D), lambda i: (i, 0)),
            pl.BlockSpec((RB, DD), lambda i: (i, 0)),
            pl.BlockSpec((1, D), lambda i: (0, 0)),
        ],
        out_specs=pl.BlockSpec((RB, D), lambda i: (i, 0)),
        out_shape=jax.ShapeDtypeStruct((N, D), jnp.float32),
    )(p, p, yp, dinv8, b.reshape(1, D))


def kernel(x, edge_index, W1, b1, Wh, bh, W2, b2):
    src = edge_index[0].astype(jnp.int32)
    dst = edge_index[1].astype(jnp.int32)
    zeros_d = jnp.zeros((K, D), jnp.float32)
    ones_d = jnp.ones((K, D), jnp.float32)

    dp = _deg_call()(dst, ones_d, zeros_d)
    y1, dinv8 = _k1(x, W1, dp)
    p = _agg_call()(y1, src, dst, zeros_d)
    y2 = _k2(p, y1, dinv8, b1, Wh)
    p = _agg_call()(y2, src, dst, zeros_d)
    y3 = _k2(p, y2, dinv8, bh, W2)
    p = _agg_call()(y3, src, dst, zeros_d)
    return _k3(p, y3, dinv8, b2)


# speedup vs baseline: 18.4035x; 1.2490x over previous
import functools

import jax
import jax.numpy as jnp
from jax import lax
from jax.experimental import pallas as pl
from jax.experimental.pallas import tpu as pltpu
from jax.experimental.pallas import tpu_sc as plsc

N = 10000
E = 320000
D = 128
DD = 16
NC = 2
NS = 16
NW = NC * NS
K = 40
EPT = E // NW
NCH = EPT // K
G = 2
NG = NCH // G
NP = 12800
RPT = NP // NS
RB = 400


def _zero_rows(buf, acc, base):
    for k in range(RPT // K):
        pltpu.sync_copy(buf, acc.at[pl.ds(base + k * K, K)])


@functools.cache
def _agg_call():
    mesh = plsc.VectorSubcoreMesh(core_axis_name="c", subcore_axis_name="s")
    scratch = (
        [pltpu.VMEM((K,), jnp.int32) for _ in range(4 * G)]
        + [pltpu.VMEM((K,), jnp.int32) for _ in range(4 * G)]
        + [pltpu.VMEM((K, D), jnp.float32) for _ in range(2 * G)]
        + [pltpu.VMEM_SHARED((NP, D), jnp.float32)]
        + [pltpu.SemaphoreType.DMA] * 8
    )

    @functools.partial(
        pl.kernel,
        mesh=mesh,
        out_type=jax.ShapeDtypeStruct((2 * NP, D), jnp.float32),
        scratch_types=scratch,
    )
    def agg(y_hbm, src_hbm, dst_hbm, z_hbm, out_hbm, *scr):
        srcs = scr[0:4 * G]
        dsts = scr[4 * G:8 * G]
        rows = scr[8 * G:10 * G]
        acc = scr[10 * G]
        isems = scr[10 * G + 1:10 * G + 5]
        gsems = scr[10 * G + 5:10 * G + 7]
        ssems = scr[10 * G + 7:10 * G + 9]
        c = lax.axis_index("c")
        s = lax.axis_index("s")
        wid = s * NC + c
        base = s * RPT
        ebase = wid * EPT

        pltpu.sync_copy(z_hbm, rows[0])
        _zero_rows(rows[0], acc, base)
        plsc.subcore_barrier()

        def fire_idx(g, ip):
            for i in range(G):
                off = ebase + (g * G + i) * K
                pltpu.async_copy(src_hbm.at[pl.ds(off, K)], srcs[ip * G + i], isems[ip])
                pltpu.async_copy(dst_hbm.at[pl.ds(off, K)], dsts[ip * G + i], isems[ip])

        def drain_idx(ip):
            for i in range(G):
                pltpu.make_async_copy(src_hbm.at[pl.ds(0, K)], srcs[ip * G + i], isems[ip]).wait()
                pltpu.make_async_copy(dst_hbm.at[pl.ds(0, K)], dsts[ip * G + i], isems[ip]).wait()

        def drain_scatters(rp):
            for i in range(G):
                pltpu.make_async_copy(y_hbm.at[pl.ds(0, K)], rows[rp * G + i], ssems[rp]).wait()

        def step(g, ip, rp, drain_prev, prefetch=True):
            if drain_prev:
                drain_scatters(rp)
            if prefetch:
                fire_idx(g + 2, (ip + 2) % 4)
            drain_idx(ip)
            for i in range(G):
                pltpu.async_copy(y_hbm.at[srcs[ip * G + i]], rows[rp * G + i], gsems[rp])
            for i in range(G):
                pltpu.make_async_copy(y_hbm.at[pl.ds(0, K)], rows[rp * G + i], gsems[rp]).wait()
            for i in range(G):
                pltpu.async_copy(rows[rp * G + i], acc.at[dsts[ip * G + i]], ssems[rp], add=True)

        fire_idx(0, 0)
        fire_idx(1, 1)
        step(0, 0, 0, drain_prev=False)
        step(1, 1, 1, drain_prev=False)
        step(2, 2, 0, drain_prev=True)
        step(3, 3, 1, drain_prev=True)

        def body(gg, carry):
            g0 = 4 * gg
            step(g0 + 0, 0, 0, drain_prev=True)
            step(g0 + 1, 1, 1, drain_prev=True)
            step(g0 + 2, 2, 0, drain_prev=True)
            step(g0 + 3, 3, 1, drain_prev=True)
            return carry

        lax.fori_loop(1, (NG - 1) // 4, body, 0, unroll=False)
        step(NG - 1, (NG - 1) % 4, (NG - 1) % 2, drain_prev=True, prefetch=False)
        drain_scatters((NG - 2) % 2)
        drain_scatters((NG - 1) % 2)
        drain_idx(NG % 4)
        plsc.subcore_barrier()
        pltpu.sync_copy(acc.at[pl.ds(base, RPT)], out_hbm.at[pl.ds(c * NP + base, RPT)])

    return agg


@functools.cache
def _deg_call():
    mesh = plsc.VectorSubcoreMesh(core_axis_name="c", subcore_axis_name="s")
    scratch = (
        [pltpu.VMEM((K,), jnp.int32) for _ in range(4 * G)]
        + [pltpu.VMEM((K, D), jnp.float32)]
        + [pltpu.VMEM_SHARED((NP, D), jnp.float32)]
        + [pltpu.SemaphoreType.DMA] * 6
    )

    @functools.partial(
        pl.kernel,
        mesh=mesh,
        out_type=jax.ShapeDtypeStruct((2 * NP, D), jnp.float32),
        scratch_types=scratch,
    )
    def deg(dst_hbm, ones_hbm, z_hbm, out_hbm, *scr):
        dsts = scr[0:4 * G]
        ones_v = scr[4 * G]
        acc = scr[4 * G + 1]
        isems = scr[4 * G + 2:4 * G + 6]
        ssems = scr[4 * G + 6:4 * G + 8]
        c = lax.axis_index("c")
        s = lax.axis_index("s")
        wid = s * NC + c
        base = s * RPT
        ebase = wid * EPT

        pltpu.sync_copy(z_hbm, ones_v)
        _zero_rows(ones_v, acc, base)
        pltpu.sync_copy(ones_hbm, ones_v)
        plsc.subcore_barrier()

        def fire_idx(g, ip):
            for i in range(G):
                off = ebase + (g * G + i) * K
                pltpu.async_copy(dst_hbm.at[pl.ds(off, K)], dsts[ip * G + i], isems[ip])

        def drain_idx(ip):
            for i in range(G):
                pltpu.make_async_copy(dst_hbm.at[pl.ds(0, K)], dsts[ip * G + i], isems[ip]).wait()

        def drain_scatters(rp):
            for _ in range(G):
                pltpu.make_async_copy(ones_hbm, ones_v, ssems[rp]).wait()

        def step(g, ip, rp, drain_prev, prefetch=True):
            if drain_prev:
                drain_scatters(rp)
            if prefetch:
                fire_idx(g + 2, (ip + 2) % 4)
            drain_idx(ip)
            for i in range(G):
                pltpu.async_copy(ones_v, acc.at[dsts[ip * G + i]], ssems[rp], add=True)

        fire_idx(0, 0)
        fire_idx(1, 1)
        step(0, 0, 0, drain_prev=False)
        step(1, 1, 1, drain_prev=False)
        step(2, 2, 0, drain_prev=True)
        step(3, 3, 1, drain_prev=True)

        def body(gg, carry):
            g0 = 4 * gg
            step(g0 + 0, 0, 0, drain_prev=True)
            step(g0 + 1, 1, 1, drain_prev=True)
            step(g0 + 2, 2, 0, drain_prev=True)
            step(g0 + 3, 3, 1, drain_prev=True)
            return carry

        lax.fori_loop(1, (NG - 1) // 4, body, 0, unroll=False)
        step(NG - 1, (NG - 1) % 4, (NG - 1) % 2, drain_prev=True, prefetch=False)
        drain_scatters((NG - 2) % 2)
        drain_scatters((NG - 1) % 2)
        drain_idx(NG % 4)
        plsc.subcore_barrier()
        pltpu.sync_copy(acc.at[pl.ds(base, RPT)], out_hbm.at[pl.ds(c * NP + base, RPT)])

    return deg


def _k1_body(x_ref, w_ref, dp0_ref, dp1_ref, y_ref, dinv_ref):
    deg = dp0_ref[...][:, 0:1] + dp1_ref[...][:, 0:1] + 1.0
    dinv = lax.rsqrt(jnp.maximum(deg, 1.0))
    xw = jnp.dot(x_ref[...], w_ref[...], preferred_element_type=jnp.float32)
    dinv_ref[...] = jnp.broadcast_to(dinv, (RB, DD))
    y_ref[...] = xw * dinv


def _k1(x, W1, dp):
    grid = (N // RB,)
    return pl.pallas_call(
        _k1_body,
        grid=grid,
        in_specs=[
            pl.BlockSpec((RB, D), lambda i: (i, 0)),
            pl.BlockSpec((D, D), lambda i: (0, 0)),
            pl.BlockSpec((RB, D), lambda i: (i, 0)),
            pl.BlockSpec((RB, D), lambda i: (i + NP // RB, 0)),
        ],
        out_specs=[
            pl.BlockSpec((RB, D), lambda i: (i, 0)),
            pl.BlockSpec((RB, DD), lambda i: (i, 0)),
        ],
        out_shape=[
            jax.ShapeDtypeStruct((N, D), jnp.float32),
            jax.ShapeDtypeStruct((N, DD), jnp.float32),
        ],
    )(x, W1, dp, dp)


def _k2_body(p0_ref, p1_ref, yp_ref, dinv_ref, b_ref, w_ref, yn_ref):
    di = dinv_ref[...][:, 0:1]
    z = (p0_ref[...] + p1_ref[...] + yp_ref[...]) * di + b_ref[...]
    h = jnp.maximum(z, 0.0)
    yn_ref[...] = jnp.dot(h, w_ref[...], preferred_element_type=jnp.float32) * di


def _k2(p, yp, dinv8, b, W):
    grid = (N // RB,)
    return pl.pallas_call(
        _k2_body,
        grid=grid,
        in_specs=[
            pl.BlockSpec((RB, D), lambda i: (i, 0)),
            pl.BlockSpec((RB, D), lambda i: (i + NP // RB, 0)),
            pl.BlockSpec((RB, D), lambda i: (i, 0)),
            pl.BlockSpec((RB, DD), lambda i: (i, 0)),
            pl.BlockSpec((1, D), lambda i: (0, 0)),
            pl.BlockSpec((D, D), lambda i: (0, 0)),
        ],
        out_specs=pl.BlockSpec((RB, D), lambda i: (i, 0)),
        out_shape=jax.ShapeDtypeStruct((N, D), jnp.float32),
    )(p, p, yp, dinv8, b.reshape(1, D), W)


def _k3_body(p0_ref, p1_ref, yp_ref, dinv_ref, b_ref, o_ref):
    di = dinv_ref[...][:, 0:1]
    z = (p0_ref[...] + p1_ref[...] + yp_ref[...]) * di + b_ref[...]
    m = jnp.max(z, axis=1, keepdims=True)
    e = jnp.exp(z - m)
    o_ref[...] = z - m - jnp.log(jnp.sum(e, axis=1, keepdims=True))


def _k3(p, yp, dinv8, b):
    grid = (N // RB,)
    return pl.pallas_call(
        _k3_body,
        grid=grid,
        in_specs=[
            pl.BlockSpec((RB, D), lambda i: (i, 0)),
            pl.BlockSpec((RB, D), lambda i: (i + NP // RB, 0)),
            pl.BlockSpec((RB, D), lambda i: (i, 0)),
            pl.BlockSpec((RB, DD), lambda i: (i, 0)),
            pl.BlockSpec((1, D), lambda i: (0, 0)),
        ],
        out_specs=pl.BlockSpec((RB, D), lambda i: (i, 0)),
        out_shape=jax.ShapeDtypeStruct((N, D), jnp.float32),
    )(p, p, yp, dinv8, b.reshape(1, D))


def kernel(x, edge_index, W1, b1, Wh, bh, W2, b2):
    pad = jnp.zeros((2 * G * K,), jnp.int32)
    src = jnp.concatenate([edge_index[0].astype(jnp.int32), pad])
    dst = jnp.concatenate([edge_index[1].astype(jnp.int32), pad])
    zeros_d = jnp.zeros((K, D), jnp.float32)
    ones_d = jnp.ones((K, D), jnp.float32)

    dp = _deg_call()(dst, ones_d, zeros_d)
    y1, dinv8 = _k1(x, W1, dp)
    p = _agg_call()(y1, src, dst, zeros_d)
    y2 = _k2(p, y1, dinv8, b1, Wh)
    p = _agg_call()(y2, src, dst, zeros_d)
    y3 = _k2(p, y2, dinv8, bh, W2)
    p = _agg_call()(y3, src, dst, zeros_d)
    return _k3(p, y3, dinv8, b2)


# speedup vs baseline: 18.4474x; 1.0024x over previous
import functools

import jax
import jax.numpy as jnp
from jax import lax
from jax.experimental import pallas as pl
from jax.experimental.pallas import tpu as pltpu
from jax.experimental.pallas import tpu_sc as plsc

N = 10000
E = 320000
D = 128
DD = 16
NC = 2
NS = 16
NW = NC * NS
K = 40
EPT = E // NW
NCH = EPT // K
G = 2
NG = NCH // G
NP = 12800
RPT = NP // NS
RB = 400


def _zero_rows(buf, acc, base):
    for k in range(RPT // K):
        pltpu.sync_copy(buf, acc.at[pl.ds(base + k * K, K)])


@functools.cache
def _agg_call():
    mesh = plsc.VectorSubcoreMesh(core_axis_name="c", subcore_axis_name="s")
    scratch = (
        [pltpu.VMEM((K,), jnp.int32) for _ in range(4 * G)]
        + [pltpu.VMEM((K,), jnp.int32) for _ in range(4 * G)]
        + [pltpu.VMEM((K, D), jnp.float32) for _ in range(2 * G)]
        + [pltpu.VMEM_SHARED((NP, D), jnp.float32)]
        + [pltpu.SemaphoreType.DMA] * 8
    )

    @functools.partial(
        pl.kernel,
        mesh=mesh,
        out_type=jax.ShapeDtypeStruct((2 * NP, D), jnp.float32),
        scratch_types=scratch,
    )
    def agg(y_hbm, src_hbm, dst_hbm, z_hbm, out_hbm, *scr):
        srcs = scr[0:4 * G]
        dsts = scr[4 * G:8 * G]
        rows = scr[8 * G:10 * G]
        acc = scr[10 * G]
        isems = scr[10 * G + 1:10 * G + 5]
        gsems = scr[10 * G + 5:10 * G + 7]
        ssems = scr[10 * G + 7:10 * G + 9]
        c = lax.axis_index("c")
        s = lax.axis_index("s")
        wid = s * NC + c
        base = s * RPT
        ebase = wid * EPT

        pltpu.sync_copy(z_hbm, rows[0])
        _zero_rows(rows[0], acc, base)
        plsc.subcore_barrier()

        def fire_idx(g, ip):
            for i in range(G):
                off = ebase + (g * G + i) * K
                pltpu.async_copy(src_hbm.at[pl.ds(off, K)], srcs[ip * G + i], isems[ip])
                pltpu.async_copy(dst_hbm.at[pl.ds(off, K)], dsts[ip * G + i], isems[ip])

        def drain_idx(ip):
            for i in range(G):
                pltpu.make_async_copy(src_hbm.at[pl.ds(0, K)], srcs[ip * G + i], isems[ip]).wait()
                pltpu.make_async_copy(dst_hbm.at[pl.ds(0, K)], dsts[ip * G + i], isems[ip]).wait()

        def drain_scatters(rp):
            for i in range(G):
                pltpu.make_async_copy(y_hbm.at[pl.ds(0, K)], rows[rp * G + i], ssems[rp]).wait()

        def step(g, ip, rp, drain_prev, prefetch=True):
            if drain_prev:
                drain_scatters(rp)
            if prefetch:
                fire_idx(g + 2, (ip + 2) % 4)
            drain_idx(ip)
            for i in range(G):
                pltpu.async_copy(y_hbm.at[srcs[ip * G + i]], rows[rp * G + i], gsems[rp])
            for i in range(G):
                pltpu.make_async_copy(y_hbm.at[pl.ds(0, K)], rows[rp * G + i], gsems[rp]).wait()
            for i in range(G):
                pltpu.async_copy(rows[rp * G + i], acc.at[dsts[ip * G + i]], ssems[rp], add=True)

        fire_idx(0, 0)
        fire_idx(1, 1)
        step(0, 0, 0, drain_prev=False)
        step(1, 1, 1, drain_prev=False)
        step(2, 2, 0, drain_prev=True)
        step(3, 3, 1, drain_prev=True)

        def body(gg, carry):
            g0 = 4 * gg
            step(g0 + 0, 0, 0, drain_prev=True)
            step(g0 + 1, 1, 1, drain_prev=True)
            step(g0 + 2, 2, 0, drain_prev=True)
            step(g0 + 3, 3, 1, drain_prev=True)
            return carry

        lax.fori_loop(1, (NG - 1) // 4, body, 0, unroll=False)
        step(NG - 1, (NG - 1) % 4, (NG - 1) % 2, drain_prev=True, prefetch=False)
        drain_scatters((NG - 2) % 2)
        drain_scatters((NG - 1) % 2)
        drain_idx(NG % 4)
        plsc.subcore_barrier()
        pltpu.sync_copy(acc.at[pl.ds(base, RPT)], out_hbm.at[pl.ds(c * NP + base, RPT)])

    return agg


@functools.cache
def _deg_call():
    mesh = plsc.VectorSubcoreMesh(core_axis_name="c", subcore_axis_name="s")
    scratch = (
        [pltpu.VMEM((K,), jnp.int32) for _ in range(4 * G)]
        + [pltpu.VMEM((K, D), jnp.float32)]
        + [pltpu.VMEM_SHARED((NP, D), jnp.float32)]
        + [pltpu.SemaphoreType.DMA] * 6
    )

    @functools.partial(
        pl.kernel,
        mesh=mesh,
        out_type=jax.ShapeDtypeStruct((2 * NP, D), jnp.float32),
        scratch_types=scratch,
    )
    def deg(dst_hbm, ones_hbm, z_hbm, out_hbm, *scr):
        dsts = scr[0:4 * G]
        ones_v = scr[4 * G]
        acc = scr[4 * G + 1]
        isems = scr[4 * G + 2:4 * G + 6]
        ssems = scr[4 * G + 6:4 * G + 8]
        c = lax.axis_index("c")
        s = lax.axis_index("s")
        wid = s * NC + c
        base = s * RPT
        ebase = wid * EPT

        pltpu.sync_copy(z_hbm, ones_v)
        _zero_rows(ones_v, acc, base)
        pltpu.sync_copy(ones_hbm, ones_v)
        plsc.subcore_barrier()

        def fire_idx(g, ip):
            for i in range(G):
                off = ebase + (g * G + i) * K
                pltpu.async_copy(dst_hbm.at[pl.ds(off, K)], dsts[ip * G + i], isems[ip])

        def drain_idx(ip):
            for i in range(G):
                pltpu.make_async_copy(dst_hbm.at[pl.ds(0, K)], dsts[ip * G + i], isems[ip]).wait()

        def drain_scatters(rp):
            for _ in range(G):
                pltpu.make_async_copy(ones_hbm, ones_v, ssems[rp]).wait()

        def step(g, ip, rp, drain_prev, prefetch=True):
            if drain_prev:
                drain_scatters(rp)
            if prefetch:
                fire_idx(g + 2, (ip + 2) % 4)
            drain_idx(ip)
            for i in range(G):
                pltpu.async_copy(ones_v, acc.at[dsts[ip * G + i]], ssems[rp], add=True)

        fire_idx(0, 0)
        fire_idx(1, 1)
        step(0, 0, 0, drain_prev=False)
        step(1, 1, 1, drain_prev=False)
        step(2, 2, 0, drain_prev=True)
        step(3, 3, 1, drain_prev=True)

        def body(gg, carry):
            g0 = 4 * gg
            step(g0 + 0, 0, 0, drain_prev=True)
            step(g0 + 1, 1, 1, drain_prev=True)
            step(g0 + 2, 2, 0, drain_prev=True)
            step(g0 + 3, 3, 1, drain_prev=True)
            return carry

        lax.fori_loop(1, (NG - 1) // 4, body, 0, unroll=False)
        step(NG - 1, (NG - 1) % 4, (NG - 1) % 2, drain_prev=True, prefetch=False)
        drain_scatters((NG - 2) % 2)
        drain_scatters((NG - 1) % 2)
        drain_idx(NG % 4)
        plsc.subcore_barrier()
        pltpu.sync_copy(acc.at[pl.ds(base, RPT)], out_hbm.at[pl.ds(c * NP + base, RPT)])

    return deg


def _k0_body(x_ref, w_ref, xw_ref):
    xw_ref[...] = jnp.dot(x_ref[...], w_ref[...], preferred_element_type=jnp.float32)


def _k0(x, W1):
    grid = (N // RB,)
    return pl.pallas_call(
        _k0_body,
        grid=grid,
        in_specs=[
            pl.BlockSpec((RB, D), lambda i: (i, 0)),
            pl.BlockSpec((D, D), lambda i: (0, 0)),
        ],
        out_specs=pl.BlockSpec((RB, D), lambda i: (i, 0)),
        out_shape=jax.ShapeDtypeStruct((N, D), jnp.float32),
    )(x, W1)


def _k1_body(xw_ref, dp0_ref, dp1_ref, y_ref, dinv_ref):
    deg = dp0_ref[...][:, 0:1] + dp1_ref[...][:, 0:1] + 1.0
    dinv = lax.rsqrt(jnp.maximum(deg, 1.0))
    dinv_ref[...] = jnp.broadcast_to(dinv, (RB, DD))
    y_ref[...] = xw_ref[...] * dinv


def _k1(xw, dp):
    grid = (N // RB,)
    return pl.pallas_call(
        _k1_body,
        grid=grid,
        in_specs=[
            pl.BlockSpec((RB, D), lambda i: (i, 0)),
            pl.BlockSpec((RB, D), lambda i: (i, 0)),
            pl.BlockSpec((RB, D), lambda i: (i + NP // RB, 0)),
        ],
        out_specs=[
            pl.BlockSpec((RB, D), lambda i: (i, 0)),
            pl.BlockSpec((RB, DD), lambda i: (i, 0)),
        ],
        out_shape=[
            jax.ShapeDtypeStruct((N, D), jnp.float32),
            jax.ShapeDtypeStruct((N, DD), jnp.float32),
        ],
    )(xw, dp, dp)


def _k2_body(p0_ref, p1_ref, yp_ref, dinv_ref, b_ref, w_ref, yn_ref):
    di = dinv_ref[...][:, 0:1]
    z = (p0_ref[...] + p1_ref[...] + yp_ref[...]) * di + b_ref[...]
    h = jnp.maximum(z, 0.0)
    yn_ref[...] = jnp.dot(h, w_ref[...], preferred_element_type=jnp.float32) * di


def _k2(p, yp, dinv8, b, W):
    grid = (N // RB,)
    return pl.pallas_call(
        _k2_body,
        grid=grid,
        in_specs=[
            pl.BlockSpec((RB, D), lambda i: (i, 0)),
            pl.BlockSpec((RB, D), lambda i: (i + NP // RB, 0)),
            pl.BlockSpec((RB, D), lambda i: (i, 0)),
            pl.BlockSpec((RB, DD), lambda i: (i, 0)),
            pl.BlockSpec((1, D), lambda i: (0, 0)),
            pl.BlockSpec((D, D), lambda i: (0, 0)),
        ],
        out_specs=pl.BlockSpec((RB, D), lambda i: (i, 0)),
        out_shape=jax.ShapeDtypeStruct((N, D), jnp.float32),
    )(p, p, yp, dinv8, b.reshape(1, D), W)


def _k3_body(p0_ref, p1_ref, yp_ref, dinv_ref, b_ref, o_ref):
    di = dinv_ref[...][:, 0:1]
    z = (p0_ref[...] + p1_ref[...] + yp_ref[...]) * di + b_ref[...]
    m = jnp.max(z, axis=1, keepdims=True)
    e = jnp.exp(z - m)
    o_ref[...] = z - m - jnp.log(jnp.sum(e, axis=1, keepdims=True))


def _k3(p, yp, dinv8, b):
    grid = (N // RB,)
    return pl.pallas_call(
        _k3_body,
        grid=grid,
        in_specs=[
            pl.BlockSpec((RB, D), lambda i: (i, 0)),
            pl.BlockSpec((RB, D), lambda i: (i + NP // RB, 0)),
            pl.BlockSpec((RB, D), lambda i: (i, 0)),
            pl.BlockSpec((RB, DD), lambda i: (i, 0)),
            pl.BlockSpec((1, D), lambda i: (0, 0)),
        ],
        out_specs=pl.BlockSpec((RB, D), lambda i: (i, 0)),
        out_shape=jax.ShapeDtypeStruct((N, D), jnp.float32),
    )(p, p, yp, dinv8, b.reshape(1, D))


def kernel(x, edge_index, W1, b1, Wh, bh, W2, b2):
    pad = jnp.zeros((2 * G * K,), jnp.int32)
    src = jnp.concatenate([edge_index[0].astype(jnp.int32), pad])
    dst = jnp.concatenate([edge_index[1].astype(jnp.int32), pad])
    zeros_d = jnp.zeros((K, D), jnp.float32)
    ones_d = jnp.ones((K, D), jnp.float32)

    dp = _deg_call()(dst, ones_d, zeros_d)
    xw = _k0(x, W1)
    y1, dinv8 = _k1(xw, dp)
    p = _agg_call()(y1, src, dst, zeros_d)
    y2 = _k2(p, y1, dinv8, b1, Wh)
    p = _agg_call()(y2, src, dst, zeros_d)
    y3 = _k2(p, y2, dinv8, bh, W2)
    p = _agg_call()(y3, src, dst, zeros_d)
    return _k3(p, y3, dinv8, b2)


# speedup vs baseline: 19.0159x; 1.0308x over previous
import functools

import jax
import jax.numpy as jnp
from jax import lax
from jax.experimental import pallas as pl
from jax.experimental.pallas import tpu as pltpu
from jax.experimental.pallas import tpu_sc as plsc

N = 10000
E = 320000
D = 128
DD = 16
NC = 2
NS = 16
NW = NC * NS
K = 40
EPT = E // NW
NCH = EPT // K
G = 2
NG = NCH // G
NP = 12800
RPT = NP // NS
RB = 1600


def _zero_rows(buf, acc, base):
    for k in range(RPT // K):
        pltpu.sync_copy(buf, acc.at[pl.ds(base + k * K, K)])


@functools.cache
def _agg_call():
    mesh = plsc.VectorSubcoreMesh(core_axis_name="c", subcore_axis_name="s")
    scratch = (
        [pltpu.VMEM((K,), jnp.int32) for _ in range(4 * G)]
        + [pltpu.VMEM((K,), jnp.int32) for _ in range(4 * G)]
        + [pltpu.VMEM((K, D), jnp.float32) for _ in range(2 * G)]
        + [pltpu.VMEM_SHARED((NP, D), jnp.float32)]
        + [pltpu.SemaphoreType.DMA] * 8
    )

    @functools.partial(
        pl.kernel,
        mesh=mesh,
        out_type=jax.ShapeDtypeStruct((2 * NP, D), jnp.float32),
        scratch_types=scratch,
    )
    def agg(y_hbm, src_hbm, dst_hbm, z_hbm, out_hbm, *scr):
        srcs = scr[0:4 * G]
        dsts = scr[4 * G:8 * G]
        rows = scr[8 * G:10 * G]
        acc = scr[10 * G]
        isems = scr[10 * G + 1:10 * G + 5]
        gsems = scr[10 * G + 5:10 * G + 7]
        ssems = scr[10 * G + 7:10 * G + 9]
        c = lax.axis_index("c")
        s = lax.axis_index("s")
        wid = s * NC + c
        base = s * RPT
        ebase = wid * EPT

        pltpu.sync_copy(z_hbm, rows[0])
        _zero_rows(rows[0], acc, base)
        plsc.subcore_barrier()

        def fire_idx(g, ip):
            for i in range(G):
                off = ebase + (g * G + i) * K
                pltpu.async_copy(src_hbm.at[pl.ds(off, K)], srcs[ip * G + i], isems[ip])
                pltpu.async_copy(dst_hbm.at[pl.ds(off, K)], dsts[ip * G + i], isems[ip])

        def drain_idx(ip):
            for i in range(G):
                pltpu.make_async_copy(src_hbm.at[pl.ds(0, K)], srcs[ip * G + i], isems[ip]).wait()
                pltpu.make_async_copy(dst_hbm.at[pl.ds(0, K)], dsts[ip * G + i], isems[ip]).wait()

        def drain_scatters(rp):
            for i in range(G):
                pltpu.make_async_copy(y_hbm.at[pl.ds(0, K)], rows[rp * G + i], ssems[rp]).wait()

        def step(g, ip, rp, drain_prev, prefetch=True):
            if drain_prev:
                drain_scatters(rp)
            if prefetch:
                fire_idx(g + 2, (ip + 2) % 4)
            drain_idx(ip)
            for i in range(G):
                pltpu.async_copy(y_hbm.at[srcs[ip * G + i]], rows[rp * G + i], gsems[rp])
            for i in range(G):
                pltpu.make_async_copy(y_hbm.at[pl.ds(0, K)], rows[rp * G + i], gsems[rp]).wait()
            for i in range(G):
                pltpu.async_copy(rows[rp * G + i], acc.at[dsts[ip * G + i]], ssems[rp], add=True)

        fire_idx(0, 0)
        fire_idx(1, 1)
        step(0, 0, 0, drain_prev=False)
        step(1, 1, 1, drain_prev=False)
        step(2, 2, 0, drain_prev=True)
        step(3, 3, 1, drain_prev=True)

        def body(gg, carry):
            g0 = 4 * gg
            step(g0 + 0, 0, 0, drain_prev=True)
            step(g0 + 1, 1, 1, drain_prev=True)
            step(g0 + 2, 2, 0, drain_prev=True)
            step(g0 + 3, 3, 1, drain_prev=True)
            return carry

        lax.fori_loop(1, (NG - 1) // 4, body, 0, unroll=False)
        step(NG - 1, (NG - 1) % 4, (NG - 1) % 2, drain_prev=True, prefetch=False)
        drain_scatters((NG - 2) % 2)
        drain_scatters((NG - 1) % 2)
        drain_idx(NG % 4)
        plsc.subcore_barrier()
        pltpu.sync_copy(acc.at[pl.ds(base, RPT)], out_hbm.at[pl.ds(c * NP + base, RPT)])

    return agg


@functools.cache
def _deg_call():
    mesh = plsc.VectorSubcoreMesh(core_axis_name="c", subcore_axis_name="s")
    scratch = (
        [pltpu.VMEM((K,), jnp.int32) for _ in range(4 * G)]
        + [pltpu.VMEM((K, D), jnp.float32)]
        + [pltpu.VMEM_SHARED((NP, D), jnp.float32)]
        + [pltpu.SemaphoreType.DMA] * 6
    )

    @functools.partial(
        pl.kernel,
        mesh=mesh,
        out_type=jax.ShapeDtypeStruct((2 * NP, D), jnp.float32),
        scratch_types=scratch,
    )
    def deg(dst_hbm, ones_hbm, z_hbm, out_hbm, *scr):
        dsts = scr[0:4 * G]
        ones_v = scr[4 * G]
        acc = scr[4 * G + 1]
        isems = scr[4 * G + 2:4 * G + 6]
        ssems = scr[4 * G + 6:4 * G + 8]
        c = lax.axis_index("c")
        s = lax.axis_index("s")
        wid = s * NC + c
        base = s * RPT
        ebase = wid * EPT

        pltpu.sync_copy(z_hbm, ones_v)
        _zero_rows(ones_v, acc, base)
        pltpu.sync_copy(ones_hbm, ones_v)
        plsc.subcore_barrier()

        def fire_idx(g, ip):
            for i in range(G):
                off = ebase + (g * G + i) * K
                pltpu.async_copy(dst_hbm.at[pl.ds(off, K)], dsts[ip * G + i], isems[ip])

        def drain_idx(ip):
            for i in range(G):
                pltpu.make_async_copy(dst_hbm.at[pl.ds(0, K)], dsts[ip * G + i], isems[ip]).wait()

        def drain_scatters(rp):
            for _ in range(G):
                pltpu.make_async_copy(ones_hbm, ones_v, ssems[rp]).wait()

        def step(g, ip, rp, drain_prev, prefetch=True):
            if drain_prev:
                drain_scatters(rp)
            if prefetch:
                fire_idx(g + 2, (ip + 2) % 4)
            drain_idx(ip)
            for i in range(G):
                pltpu.async_copy(ones_v, acc.at[dsts[ip * G + i]], ssems[rp], add=True)

        fire_idx(0, 0)
        fire_idx(1, 1)
        step(0, 0, 0, drain_prev=False)
        step(1, 1, 1, drain_prev=False)
        step(2, 2, 0, drain_prev=True)
        step(3, 3, 1, drain_prev=True)

        def body(gg, carry):
            g0 = 4 * gg
            step(g0 + 0, 0, 0, drain_prev=True)
            step(g0 + 1, 1, 1, drain_prev=True)
            step(g0 + 2, 2, 0, drain_prev=True)
            step(g0 + 3, 3, 1, drain_prev=True)
            return carry

        lax.fori_loop(1, (NG - 1) // 4, body, 0, unroll=False)
        step(NG - 1, (NG - 1) % 4, (NG - 1) % 2, drain_prev=True, prefetch=False)
        drain_scatters((NG - 2) % 2)
        drain_scatters((NG - 1) % 2)
        drain_idx(NG % 4)
        plsc.subcore_barrier()
        pltpu.sync_copy(acc.at[pl.ds(base, RPT)], out_hbm.at[pl.ds(c * NP + base, RPT)])

    return deg


def _k0_body(x_ref, w_ref, xw_ref):
    xw_ref[...] = jnp.dot(x_ref[...], w_ref[...], preferred_element_type=jnp.float32)


def _k0(x, W1):
    grid = (NP // RB,)
    return pl.pallas_call(
        _k0_body,
        grid=grid,
        in_specs=[
            pl.BlockSpec((RB, D), lambda i: (i, 0)),
            pl.BlockSpec((D, D), lambda i: (0, 0)),
        ],
        out_specs=pl.BlockSpec((RB, D), lambda i: (i, 0)),
        out_shape=jax.ShapeDtypeStruct((NP, D), jnp.float32),
    )(x, W1)


def _k1_body(xw_ref, dp0_ref, dp1_ref, y_ref, dinv_ref):
    deg = dp0_ref[...][:, 0:1] + dp1_ref[...][:, 0:1] + 1.0
    dinv = lax.rsqrt(jnp.maximum(deg, 1.0))
    dinv_ref[...] = jnp.broadcast_to(dinv, (RB, DD))
    y_ref[...] = xw_ref[...] * dinv


def _k1(xw, dp):
    grid = (NP // RB,)
    return pl.pallas_call(
        _k1_body,
        grid=grid,
        in_specs=[
            pl.BlockSpec((RB, D), lambda i: (i, 0)),
            pl.BlockSpec((RB, D), lambda i: (i, 0)),
            pl.BlockSpec((RB, D), lambda i: (i + NP // RB, 0)),
        ],
        out_specs=[
            pl.BlockSpec((RB, D), lambda i: (i, 0)),
            pl.BlockSpec((RB, DD), lambda i: (i, 0)),
        ],
        out_shape=[
            jax.ShapeDtypeStruct((NP, D), jnp.float32),
            jax.ShapeDtypeStruct((NP, DD), jnp.float32),
        ],
    )(xw, dp, dp)


def _k2_body(p0_ref, p1_ref, yp_ref, dinv_ref, b_ref, w_ref, yn_ref):
    di = dinv_ref[...][:, 0:1]
    z = (p0_ref[...] + p1_ref[...] + yp_ref[...]) * di + b_ref[...]
    h = jnp.maximum(z, 0.0)
    yn_ref[...] = jnp.dot(h, w_ref[...], preferred_element_type=jnp.float32) * di


def _k2(p, yp, dinv8, b, W):
    grid = (NP // RB,)
    return pl.pallas_call(
        _k2_body,
        grid=grid,
        in_specs=[
            pl.BlockSpec((RB, D), lambda i: (i, 0)),
            pl.BlockSpec((RB, D), lambda i: (i + NP // RB, 0)),
            pl.BlockSpec((RB, D), lambda i: (i, 0)),
            pl.BlockSpec((RB, DD), lambda i: (i, 0)),
            pl.BlockSpec((1, D), lambda i: (0, 0)),
            pl.BlockSpec((D, D), lambda i: (0, 0)),
        ],
        out_specs=pl.BlockSpec((RB, D), lambda i: (i, 0)),
        out_shape=jax.ShapeDtypeStruct((NP, D), jnp.float32),
    )(p, p, yp, dinv8, b.reshape(1, D), W)


def _k3_body(p0_ref, p1_ref, yp_ref, dinv_ref, b_ref, o_ref):
    di = dinv_ref[...][:, 0:1]
    z = (p0_ref[...] + p1_ref[...] + yp_ref[...]) * di + b_ref[...]
    m = jnp.max(z, axis=1, keepdims=True)
    e = jnp.exp(z - m)
    o_ref[...] = z - m - jnp.log(jnp.sum(e, axis=1, keepdims=True))


def _k3(p, yp, dinv8, b):
    grid = (NP // RB,)
    return pl.pallas_call(
        _k3_body,
        grid=grid,
        in_specs=[
            pl.BlockSpec((RB, D), lambda i: (i, 0)),
            pl.BlockSpec((RB, D), lambda i: (i + NP // RB, 0)),
            pl.BlockSpec((RB, D), lambda i: (i, 0)),
            pl.BlockSpec((RB, DD), lambda i: (i, 0)),
            pl.BlockSpec((1, D), lambda i: (0, 0)),
        ],
        out_specs=pl.BlockSpec((RB, D), lambda i: (i, 0)),
        out_shape=jax.ShapeDtypeStruct((NP, D), jnp.float32),
    )(p, p, yp, dinv8, b.reshape(1, D))


def kernel(x, edge_index, W1, b1, Wh, bh, W2, b2):
    pad = jnp.zeros((2 * G * K,), jnp.int32)
    src = jnp.concatenate([edge_index[0].astype(jnp.int32), pad])
    dst = jnp.concatenate([edge_index[1].astype(jnp.int32), pad])
    zeros_d = jnp.zeros((K, D), jnp.float32)
    ones_d = jnp.ones((K, D), jnp.float32)

    xp = jnp.concatenate([x, jnp.zeros((NP - N, D), jnp.float32)])
    dp = _deg_call()(dst, ones_d, zeros_d)
    xw = _k0(xp, W1)
    y1, dinv8 = _k1(xw, dp)
    p = _agg_call()(y1, src, dst, zeros_d)
    y2 = _k2(p, y1, dinv8, b1, Wh)
    p = _agg_call()(y2, src, dst, zeros_d)
    y3 = _k2(p, y2, dinv8, bh, W2)
    p = _agg_call()(y3, src, dst, zeros_d)
    return _k3(p, y3, dinv8, b2)[:N]
